# Initial kernel scaffold; baseline (speedup 1.0000x reference)
import jax
import jax.numpy as jnp
from jax.experimental import pallas as pl


def kernel(pred_logits, points, knn_indices, gt_triangles):
    raise NotImplementedError("write your pallas kernel here")


# speedup vs baseline: 12.3403x; 12.3403x over previous
import jax
import jax.numpy as jnp
from jax.experimental import pallas as pl

_NP = 2048
_K = 8
_C = (_K - 1) * (_K - 1)
_BLK = 128
_GRID = _NP // _BLK


def _prep_kernel(gt_ref, gt_t_ref, tri_ref, valid_ref, ekeys_ref):
    a = gt_t_ref[0:1, :]
    b = gt_t_ref[1:2, :]
    c = gt_t_ref[2:3, :]
    p = jnp.minimum(a, b)
    q = jnp.maximum(a, b)
    s0 = jnp.minimum(p, c)
    s2 = jnp.maximum(q, c)
    s1 = a + b + c - s0 - s2
    tri_ref[0:1, :] = s0
    tri_ref[1:2, :] = s1
    tri_ref[2:3, :] = s2
    ekeys_ref[0:1, :] = s0 * _NP + s1
    ekeys_ref[1:2, :] = s1 * _NP + s2
    ekeys_ref[2:3, :] = s0 * _NP + s2
    khi_r = s0 * _NP + s1
    klo_r = s2

    ac = gt_ref[:, 0:1]
    bc = gt_ref[:, 1:2]
    cc = gt_ref[:, 2:3]
    pc = jnp.minimum(ac, bc)
    qc = jnp.maximum(ac, bc)
    s0c = jnp.minimum(pc, cc)
    s2c = jnp.maximum(qc, cc)
    s1c = ac + bc + cc - s0c - s2c
    khi_c = s0c * _NP + s1c
    klo_c = s2c

    u_lane = jax.lax.broadcasted_iota(jnp.int32, (_BLK, _NP), 1)
    t_sub = jax.lax.broadcasted_iota(jnp.int32, (_BLK, _NP), 0)
    for ch in range(_NP // _BLK):
        beg, end = ch * _BLK, (ch + 1) * _BLK
        eq = (khi_c[beg:end, :] == khi_r) & (klo_c[beg:end, :] == klo_r)
        dup = jnp.any(eq & (u_lane < (t_sub + ch * _BLK)), axis=1,
                      keepdims=True)
        valid_ref[beg:end, :] = jnp.where(dup, 0, 1).astype(jnp.int32)


def _main_kernel(pred_ref, pred_t_ref, knn_ref, tri_ref, valid_ref,
                 trihi3_ref, trilo3_ref, ekeys3_ref,
                 sbce_ref, sman_ref, spos_ref, sneg_ref, sm_ref):
    step = pl.program_id(0)

    zero11 = jnp.zeros((1, 1), dtype=jnp.float32)

    @pl.when(step == 0)
    def _init():
        sbce_ref[:, :] = zero11
        sman_ref[:, :] = zero11
        spos_ref[:, :] = zero11
        sneg_ref[:, :] = zero11
        sm_ref[:, :] = zero11

    v0 = knn_ref[:, 0:1]
    nn_cols = [knn_ref[:, 1 + j:2 + j] for j in range(_K - 1)]

    cand_a = jnp.concatenate(
        [nn_cols[j] for j in range(_K - 1) for _ in range(_K - 1)], axis=1)
    cand_b = jnp.concatenate(
        [nn_cols[k] for _ in range(_K - 1) for k in range(_K - 1)], axis=1)

    lo = jnp.minimum(cand_a, cand_b)
    hi = jnp.maximum(cand_a, cand_b)
    ekey_cand = lo * _NP + hi

    nn_rest = knn_ref[:, 1:_K]
    v0b7 = jnp.broadcast_to(v0, nn_rest.shape)
    clo7 = jnp.minimum(v0b7, nn_rest)
    chi7 = jnp.maximum(v0b7, nn_rest)
    ekey_cent = clo7 * _NP + chi7

    ekq = jnp.concatenate([ekey_cent, ekey_cand], axis=1)
    lab_acc = jnp.zeros(ekq.shape, dtype=jnp.bool_)
    n_ek = 3 * _NP
    ek_ch = 256
    for ch in range(n_ek // ek_ch):
        ek = ekeys3_ref[0:1, 0:1, pl.ds(ch * ek_ch, ek_ch)]
        hitc = jnp.any(ekq[:, :, None] == ek, axis=2)
        lab_acc = lab_acc | hitc
    labels56 = lab_acc.astype(jnp.float32)
    gl = labels56[:, : _K - 1]
    gt_labels = labels56[:, _K - 1:]

    v0b = jnp.broadcast_to(v0, cand_a.shape)
    c0 = jnp.minimum(lo, v0b)
    c2 = jnp.maximum(hi, v0b)
    c1 = cand_a + cand_b + v0b - c0 - c2
    tk_hi = c0 * _NP + c1
    tk_lo = c2

    m_acc = jnp.zeros(tk_hi.shape, dtype=jnp.bool_)
    t_ch = 256
    for ch in range(_NP // t_ch):
        th = trihi3_ref[0:1, 0:1, pl.ds(ch * t_ch, t_ch)]
        tl = trilo3_ref[0:1, 0:1, pl.ds(ch * t_ch, t_ch)]
        eq = (tk_hi[:, :, None] == th) & (tk_lo[:, :, None] == tl)
        m_acc = m_acc | jnp.any(eq, axis=2)
    gt_mask = m_acc.astype(jnp.float32)

    s0r = tri_ref[0:1, :]
    s1r = tri_ref[1:2, :]
    s2r = tri_ref[2:3, :]
    vrow = valid_ref[0:1, :]
    contains = ((v0 == s0r) | (v0 == s1r) | (v0 == s2r)) & (vrow > 0)
    angt = jnp.sum(contains.astype(jnp.float32), axis=1, keepdims=True)

    glm = gt_labels * gt_mask
    npred = jnp.sum(glm, axis=1, keepdims=True)
    manifold = (2.0 * angt == npred).astype(jnp.float32)

    x = pred_ref[:, :]
    bce = jnp.maximum(x, 0.0) - x * glm + jnp.log(1.0 + jnp.exp(-jnp.abs(x)))
    sbce = jnp.sum(bce * manifold, keepdims=True).reshape(1, 1)
    sman = jnp.sum(manifold, keepdims=True).reshape(1, 1)

    xk = [pred_t_ref[k] for k in range(_K - 1)]
    pos = jnp.zeros(xk[0].shape, dtype=jnp.float32)
    neg = jnp.zeros(xk[0].shape, dtype=jnp.float32)
    for k in range(_K - 1):
        cnt = jnp.zeros(xk[0].shape, dtype=jnp.int32)
        for kp in range(_K - 1):
            if kp == k:
                continue
            gtk = (xk[kp] > xk[k]) | ((xk[kp] == xk[k]) & (kp < k))
            cnt = cnt + gtk.astype(jnp.int32)
        pos = pos + xk[k] * (cnt == 1).astype(jnp.float32)
        neg = neg + xk[k] * (cnt == 2).astype(jnp.float32)
    mmask = (gl == 1.0).astype(jnp.float32)
    bpos = jnp.maximum(pos, 0.0) - pos + jnp.log(1.0 + jnp.exp(-jnp.abs(pos)))
    bneg = jnp.maximum(neg, 0.0) + jnp.log(1.0 + jnp.exp(-jnp.abs(neg)))
    spos = jnp.sum(bpos * mmask, keepdims=True).reshape(1, 1)
    sneg = jnp.sum(bneg * mmask, keepdims=True).reshape(1, 1)
    sm = jnp.sum(mmask, keepdims=True).reshape(1, 1)

    sbce_ref[:, :] += sbce
    sman_ref[:, :] += sman
    spos_ref[:, :] += spos
    sneg_ref[:, :] += sneg
    sm_ref[:, :] += sm


def kernel(pred_logits, points, knn_indices, gt_triangles):
    del points
    gt = gt_triangles.astype(jnp.int32)
    knn = knn_indices.astype(jnp.int32)
    gt_t = gt.T

    tri, valid_c, ekeys = pl.pallas_call(
        _prep_kernel,
        out_shape=[
            jax.ShapeDtypeStruct((3, _NP), jnp.int32),
            jax.ShapeDtypeStruct((_NP, 1), jnp.int32),
            jax.ShapeDtypeStruct((3, _NP), jnp.int32),
        ],
    )(gt, gt_t)

    valid = valid_c.reshape(1, _NP)
    trihi3 = (tri[0] * _NP + tri[1]).reshape(1, 1, _NP)
    trilo3 = tri[2].reshape(1, 1, _NP)
    ekeys3 = ekeys.reshape(1, 1, 3 * _NP)
    pred_t = pred_logits.reshape(_NP, _K - 1, _K - 1).transpose(2, 0, 1)

    outs = pl.pallas_call(
        _main_kernel,
        grid=(_GRID,),
        in_specs=[
            pl.BlockSpec((_BLK, _C), lambda i: (i, 0)),
            pl.BlockSpec((_K - 1, _BLK, _K - 1), lambda i: (0, i, 0)),
            pl.BlockSpec((_BLK, _K), lambda i: (i, 0)),
            pl.BlockSpec((3, _NP), lambda i: (0, 0)),
            pl.BlockSpec((1, _NP), lambda i: (0, 0)),
            pl.BlockSpec((1, 1, _NP), lambda i: (0, 0, 0)),
            pl.BlockSpec((1, 1, _NP), lambda i: (0, 0, 0)),
            pl.BlockSpec((1, 1, 3 * _NP), lambda i: (0, 0, 0)),
        ],
        out_specs=[pl.BlockSpec((1, 1), lambda i: (0, 0))] * 5,
        out_shape=[jax.ShapeDtypeStruct((1, 1), jnp.float32)] * 5,
    )(pred_logits, pred_t, knn, tri, valid, trihi3, trilo3, ekeys3)

    sbce, sman, spos, sneg, sm = [o[0, 0] for o in outs]
    loss = sbce / (_C * sman)
    loss_pos = spos / sm
    loss_neg = sneg / sm
    return (loss, loss_pos, loss_neg)


# speedup vs baseline: 40.5481x; 3.2858x over previous
import jax
import jax.numpy as jnp
from jax.experimental import pallas as pl

_NP = 2048
_K = 8
_C = (_K - 1) * (_K - 1)
_BLK = 128
_GRID = _NP // _BLK


def _prep_kernel(gt_ref, gt_t_ref, tri_ref, valid_ref, ekeys_ref):
    a = gt_t_ref[0:1, :]
    b = gt_t_ref[1:2, :]
    c = gt_t_ref[2:3, :]
    p = jnp.minimum(a, b)
    q = jnp.maximum(a, b)
    s0 = jnp.minimum(p, c)
    s2 = jnp.maximum(q, c)
    s1 = a + b + c - s0 - s2
    tri_ref[0:1, :] = s0
    tri_ref[1:2, :] = s1
    tri_ref[2:3, :] = s2
    ekeys_ref[0:1, :] = s0 * _NP + s1
    ekeys_ref[1:2, :] = s1 * _NP + s2
    ekeys_ref[2:3, :] = s0 * _NP + s2
    khi_r = s0 * _NP + s1
    klo_r = s2

    ac = gt_ref[:, 0:1]
    bc = gt_ref[:, 1:2]
    cc = gt_ref[:, 2:3]
    pc = jnp.minimum(ac, bc)
    qc = jnp.maximum(ac, bc)
    s0c = jnp.minimum(pc, cc)
    s2c = jnp.maximum(qc, cc)
    s1c = ac + bc + cc - s0c - s2c
    khi_c = s0c * _NP + s1c
    klo_c = s2c

    u_lane = jax.lax.broadcasted_iota(jnp.int32, (_BLK, _NP), 1)
    t_sub = jax.lax.broadcasted_iota(jnp.int32, (_BLK, _NP), 0)
    for ch in range(_NP // _BLK):
        beg, end = ch * _BLK, (ch + 1) * _BLK
        eq = (khi_c[beg:end, :] == khi_r) & (klo_c[beg:end, :] == klo_r)
        dup = jnp.any(eq & (u_lane < (t_sub + ch * _BLK)), axis=1,
                      keepdims=True)
        valid_ref[beg:end, :] = jnp.where(dup, 0, 1).astype(jnp.int32)


def _main_kernel(pred_ref, pred_t_ref, knn_ref, tri_ref, valid_ref,
                 trihi3_ref, trilo3_ref, ekeys3_ref,
                 sbce_ref, sman_ref, spos_ref, sneg_ref, sm_ref):
    step = pl.program_id(0)

    zero11 = jnp.zeros((1, 1), dtype=jnp.float32)

    @pl.when(step == 0)
    def _init():
        sbce_ref[:, :] = zero11
        sman_ref[:, :] = zero11
        spos_ref[:, :] = zero11
        sneg_ref[:, :] = zero11
        sm_ref[:, :] = zero11

    v0 = knn_ref[:, 0:1]
    nn_cols = [knn_ref[:, 1 + j:2 + j] for j in range(_K - 1)]

    pairs = [(j, k) for j in range(_K - 1) for k in range(j, _K - 1)]

    nn_rest = knn_ref[:, 1:_K]
    v0b7 = jnp.broadcast_to(v0, nn_rest.shape)
    clo7 = jnp.minimum(v0b7, nn_rest)
    chi7 = jnp.maximum(v0b7, nn_rest)
    ekq = clo7 * _NP + chi7

    lab_acc = jnp.zeros(ekq.shape, dtype=jnp.bool_)
    n_ek = 3 * _NP
    ek_ch = 256
    for ch in range(n_ek // ek_ch):
        ek = ekeys3_ref[0:1, 0:1, pl.ds(ch * ek_ch, ek_ch)]
        hitc = jnp.any(ekq[:, :, None] == ek, axis=2)
        lab_acc = lab_acc | hitc
    gl = lab_acc.astype(jnp.float32)

    cand_a = jnp.concatenate([nn_cols[j] for j, _ in pairs], axis=1)
    cand_b = jnp.concatenate([nn_cols[k] for _, k in pairs], axis=1)
    lo = jnp.minimum(cand_a, cand_b)
    hi = jnp.maximum(cand_a, cand_b)
    v0b = jnp.broadcast_to(v0, cand_a.shape)
    c0 = jnp.minimum(lo, v0b)
    c2 = jnp.maximum(hi, v0b)
    c1 = cand_a + cand_b + v0b - c0 - c2
    tk_hi = c0 * _NP + c1
    tk_lo = c2

    m_acc = jnp.zeros(tk_hi.shape, dtype=jnp.bool_)
    t_ch = 256
    for ch in range(_NP // t_ch):
        th = trihi3_ref[0:1, 0:1, pl.ds(ch * t_ch, t_ch)]
        tl = trilo3_ref[0:1, 0:1, pl.ds(ch * t_ch, t_ch)]
        eq = (tk_hi[:, :, None] == th) & (tk_lo[:, :, None] == tl)
        m_acc = m_acc | jnp.any(eq, axis=2)
    mask28 = m_acc.astype(jnp.float32)

    s0r = tri_ref[0:1, :]
    s1r = tri_ref[1:2, :]
    s2r = tri_ref[2:3, :]
    vrow = valid_ref[0:1, :]
    contains = ((v0 == s0r) | (v0 == s1r) | (v0 == s2r)) & (vrow > 0)
    angt = jnp.sum(contains.astype(jnp.float32), axis=1, keepdims=True)

    wrow = jnp.concatenate(
        [jnp.full((1, 1), 1.0 if j == k else 2.0, jnp.float32)
         for j, k in pairs], axis=1)
    npred = jnp.sum(mask28 * wrow, axis=1, keepdims=True)
    manifold = (2.0 * angt == npred).astype(jnp.float32)

    x = pred_ref[:, :]
    bce0 = jnp.maximum(x, 0.0) + jnp.log(1.0 + jnp.exp(-jnp.abs(x)))
    xsym = jnp.concatenate(
        [x[:, j * (_K - 1) + k:j * (_K - 1) + k + 1]
         if j == k else
         (x[:, j * (_K - 1) + k:j * (_K - 1) + k + 1]
          + x[:, k * (_K - 1) + j:k * (_K - 1) + j + 1])
         for j, k in pairs], axis=1)
    row_bce = (jnp.sum(bce0, axis=1, keepdims=True)
               - jnp.sum(xsym * mask28, axis=1, keepdims=True))
    sbce = jnp.sum(row_bce * manifold, keepdims=True).reshape(1, 1)
    sman = jnp.sum(manifold, keepdims=True).reshape(1, 1)

    xk = [pred_t_ref[k] for k in range(_K - 1)]
    pos = jnp.zeros(xk[0].shape, dtype=jnp.float32)
    neg = jnp.zeros(xk[0].shape, dtype=jnp.float32)
    for k in range(_K - 1):
        cnt = jnp.zeros(xk[0].shape, dtype=jnp.int32)
        for kp in range(_K - 1):
            if kp == k:
                continue
            gtk = (xk[kp] > xk[k]) | ((xk[kp] == xk[k]) & (kp < k))
            cnt = cnt + gtk.astype(jnp.int32)
        pos = pos + xk[k] * (cnt == 1).astype(jnp.float32)
        neg = neg + xk[k] * (cnt == 2).astype(jnp.float32)
    mmask = (gl == 1.0).astype(jnp.float32)
    bpos = jnp.maximum(pos, 0.0) - pos + jnp.log(1.0 + jnp.exp(-jnp.abs(pos)))
    bneg = jnp.maximum(neg, 0.0) + jnp.log(1.0 + jnp.exp(-jnp.abs(neg)))
    spos = jnp.sum(bpos * mmask, keepdims=True).reshape(1, 1)
    sneg = jnp.sum(bneg * mmask, keepdims=True).reshape(1, 1)
    sm = jnp.sum(mmask, keepdims=True).reshape(1, 1)

    sbce_ref[:, :] += sbce
    sman_ref[:, :] += sman
    spos_ref[:, :] += spos
    sneg_ref[:, :] += sneg
    sm_ref[:, :] += sm


def kernel(pred_logits, points, knn_indices, gt_triangles):
    del points
    gt = gt_triangles.astype(jnp.int32)
    knn = knn_indices.astype(jnp.int32)
    gt_t = gt.T

    tri, valid_c, ekeys = pl.pallas_call(
        _prep_kernel,
        out_shape=[
            jax.ShapeDtypeStruct((3, _NP), jnp.int32),
            jax.ShapeDtypeStruct((_NP, 1), jnp.int32),
            jax.ShapeDtypeStruct((3, _NP), jnp.int32),
        ],
    )(gt, gt_t)

    valid = valid_c.reshape(1, _NP)
    trihi3 = (tri[0] * _NP + tri[1]).reshape(1, 1, _NP)
    trilo3 = tri[2].reshape(1, 1, _NP)
    ekeys3 = ekeys.reshape(1, 1, 3 * _NP)
    pred_t = pred_logits.reshape(_NP, _K - 1, _K - 1).transpose(2, 0, 1)

    outs = pl.pallas_call(
        _main_kernel,
        grid=(_GRID,),
        in_specs=[
            pl.BlockSpec((_BLK, _C), lambda i: (i, 0)),
            pl.BlockSpec((_K - 1, _BLK, _K - 1), lambda i: (0, i, 0)),
            pl.BlockSpec((_BLK, _K), lambda i: (i, 0)),
            pl.BlockSpec((3, _NP), lambda i: (0, 0)),
            pl.BlockSpec((1, _NP), lambda i: (0, 0)),
            pl.BlockSpec((1, 1, _NP), lambda i: (0, 0, 0)),
            pl.BlockSpec((1, 1, _NP), lambda i: (0, 0, 0)),
            pl.BlockSpec((1, 1, 3 * _NP), lambda i: (0, 0, 0)),
        ],
        out_specs=[pl.BlockSpec((1, 1), lambda i: (0, 0))] * 5,
        out_shape=[jax.ShapeDtypeStruct((1, 1), jnp.float32)] * 5,
    )(pred_logits, pred_t, knn, tri, valid, trihi3, trilo3, ekeys3)

    sbce, sman, spos, sneg, sm = [o[0, 0] for o in outs]
    loss = sbce / (_C * sman)
    loss_pos = spos / sm
    loss_neg = sneg / sm
    return (loss, loss_pos, loss_neg)


# speedup vs baseline: 43.0249x; 1.0611x over previous
import jax
import jax.numpy as jnp
from jax.experimental import pallas as pl
from jax.experimental.pallas import tpu as pltpu

_NP = 2048
_K = 8
_C = (_K - 1) * (_K - 1)
_BLK = 128
_GRID = _NP // _BLK


def _main_kernel(pred_ref, pred_t_ref, knn_ref, gt_ref, gt_t_ref,
                 sbce_ref, sman_ref, spos_ref, sneg_ref, sm_ref,
                 tri_ref, hi3_ref, lo3_ref, valid_ref):
    step = pl.program_id(0)
    zero11 = jnp.zeros((1, 1), dtype=jnp.float32)

    @pl.when(step == 0)
    def _prep():
        a = gt_t_ref[0:1, :]
        b = gt_t_ref[1:2, :]
        c = gt_t_ref[2:3, :]
        p = jnp.minimum(a, b)
        q = jnp.maximum(a, b)
        s0 = jnp.minimum(p, c)
        s2 = jnp.maximum(q, c)
        s1 = a + b + c - s0 - s2
        tri_ref[0:1, :] = s0
        tri_ref[1:2, :] = s1
        tri_ref[2:3, :] = s2
        khi_r = s0 * _NP + s1
        klo_r = s2
        hi3_ref[0:1, 0:1, :] = khi_r.reshape(1, 1, _NP)
        lo3_ref[0:1, 0:1, :] = klo_r.reshape(1, 1, _NP)

        ac = gt_ref[:, 0:1]
        bc = gt_ref[:, 1:2]
        cc = gt_ref[:, 2:3]
        pc = jnp.minimum(ac, bc)
        qc = jnp.maximum(ac, bc)
        s0c = jnp.minimum(pc, cc)
        s2c = jnp.maximum(qc, cc)
        s1c = ac + bc + cc - s0c - s2c
        khi_c = s0c * _NP + s1c
        klo_c = s2c

        u_lane = jax.lax.broadcasted_iota(jnp.int32, (_BLK, _NP), 1)
        t_sub = jax.lax.broadcasted_iota(jnp.int32, (_BLK, _NP), 0)
        for ch in range(_NP // _BLK):
            beg, end = ch * _BLK, (ch + 1) * _BLK
            eq = (khi_c[beg:end, :] == khi_r) & (klo_c[beg:end, :] == klo_r)
            dup = jnp.any(eq & (u_lane < (t_sub + ch * _BLK)), axis=1,
                          keepdims=True)
            valid_ref[0:1, beg:end] = jnp.where(
                dup, 0, 1).astype(jnp.int32).reshape(1, _BLK)

        sbce_ref[:, :] = zero11
        sman_ref[:, :] = zero11
        spos_ref[:, :] = zero11
        sneg_ref[:, :] = zero11
        sm_ref[:, :] = zero11

    v0 = knn_ref[:, 0:1]
    nn_cols = [knn_ref[:, 1 + j:2 + j] for j in range(_K - 1)]
    pairs = [(j, k) for j in range(_K - 1) for k in range(j, _K - 1)]

    s0r = tri_ref[0:1, :]
    s1r = tri_ref[1:2, :]
    s2r = tri_ref[2:3, :]
    vrow = valid_ref[0:1, :]

    e0 = v0 == s0r
    e1 = v0 == s1r
    e2 = v0 == s2r
    in0 = e0 | e1 | e2

    contains = in0 & (vrow > 0)
    angt = jnp.sum(contains.astype(jnp.float32), axis=1, keepdims=True)

    dup_any = jnp.max(
        jnp.where((e0 & e1) | (e1 & e2), 1.0, 0.0), axis=1, keepdims=True)
    gl_cols = []
    for j in range(_K - 1):
        nj = nn_cols[j]
        inj = (nj == s0r) | (nj == s1r) | (nj == s2r)
        both = jnp.max(
            jnp.where(in0 & inj, 1.0, 0.0), axis=1, keepdims=True)
        gl_cols.append(jnp.where(nj == v0, dup_any, both))
    gl = jnp.concatenate(gl_cols, axis=1)

    cand_a = jnp.concatenate([nn_cols[j] for j, _ in pairs], axis=1)
    cand_b = jnp.concatenate([nn_cols[k] for _, k in pairs], axis=1)
    lo = jnp.minimum(cand_a, cand_b)
    hi = jnp.maximum(cand_a, cand_b)
    v0b = jnp.broadcast_to(v0, cand_a.shape)
    c0 = jnp.minimum(lo, v0b)
    c2 = jnp.maximum(hi, v0b)
    c1 = cand_a + cand_b + v0b - c0 - c2
    tk_hi = c0 * _NP + c1
    tk_lo = c2

    m_acc = jnp.zeros(tk_hi.shape, dtype=jnp.bool_)
    t_ch = 256
    for ch in range(_NP // t_ch):
        th = hi3_ref[0:1, 0:1, pl.ds(ch * t_ch, t_ch)]
        tl = lo3_ref[0:1, 0:1, pl.ds(ch * t_ch, t_ch)]
        eq = (tk_hi[:, :, None] == th) & (tk_lo[:, :, None] == tl)
        m_acc = m_acc | jnp.any(eq, axis=2)
    mask28 = m_acc.astype(jnp.float32)

    wrow = jnp.concatenate(
        [jnp.full((1, 1), 1.0 if j == k else 2.0, jnp.float32)
         for j, k in pairs], axis=1)
    npred = jnp.sum(mask28 * wrow, axis=1, keepdims=True)
    manifold = (2.0 * angt == npred).astype(jnp.float32)

    x = pred_ref[:, :]
    bce0 = jnp.maximum(x, 0.0) + jnp.log(1.0 + jnp.exp(-jnp.abs(x)))
    xsym = jnp.concatenate(
        [x[:, j * (_K - 1) + k:j * (_K - 1) + k + 1]
         if j == k else
         (x[:, j * (_K - 1) + k:j * (_K - 1) + k + 1]
          + x[:, k * (_K - 1) + j:k * (_K - 1) + j + 1])
         for j, k in pairs], axis=1)
    row_bce = (jnp.sum(bce0, axis=1, keepdims=True)
               - jnp.sum(xsym * mask28, axis=1, keepdims=True))
    sbce = jnp.sum(row_bce * manifold, keepdims=True).reshape(1, 1)
    sman = jnp.sum(manifold, keepdims=True).reshape(1, 1)

    xk = [pred_t_ref[k] for k in range(_K - 1)]
    pos = jnp.zeros(xk[0].shape, dtype=jnp.float32)
    neg = jnp.zeros(xk[0].shape, dtype=jnp.float32)
    for k in range(_K - 1):
        cnt = jnp.zeros(xk[0].shape, dtype=jnp.int32)
        for kp in range(_K - 1):
            if kp == k:
                continue
            gtk = (xk[kp] > xk[k]) | ((xk[kp] == xk[k]) & (kp < k))
            cnt = cnt + gtk.astype(jnp.int32)
        pos = pos + xk[k] * (cnt == 1).astype(jnp.float32)
        neg = neg + xk[k] * (cnt == 2).astype(jnp.float32)
    mmask = (gl == 1.0).astype(jnp.float32)
    bpos = jnp.maximum(pos, 0.0) - pos + jnp.log(1.0 + jnp.exp(-jnp.abs(pos)))
    bneg = jnp.maximum(neg, 0.0) + jnp.log(1.0 + jnp.exp(-jnp.abs(neg)))
    spos = jnp.sum(bpos * mmask, keepdims=True).reshape(1, 1)
    sneg = jnp.sum(bneg * mmask, keepdims=True).reshape(1, 1)
    sm = jnp.sum(mmask, keepdims=True).reshape(1, 1)

    sbce_ref[:, :] += sbce
    sman_ref[:, :] += sman
    spos_ref[:, :] += spos
    sneg_ref[:, :] += sneg
    sm_ref[:, :] += sm


def kernel(pred_logits, points, knn_indices, gt_triangles):
    del points
    gt = gt_triangles.astype(jnp.int32)
    knn = knn_indices.astype(jnp.int32)
    gt_t = gt.T
    pred_t = pred_logits.reshape(_NP, _K - 1, _K - 1).transpose(2, 0, 1)

    outs = pl.pallas_call(
        _main_kernel,
        grid=(_GRID,),
        in_specs=[
            pl.BlockSpec((_BLK, _C), lambda i: (i, 0)),
            pl.BlockSpec((_K - 1, _BLK, _K - 1), lambda i: (0, i, 0)),
            pl.BlockSpec((_BLK, _K), lambda i: (i, 0)),
            pl.BlockSpec((_NP, 3), lambda i: (0, 0)),
            pl.BlockSpec((3, _NP), lambda i: (0, 0)),
        ],
        out_specs=[pl.BlockSpec((1, 1), lambda i: (0, 0))] * 5,
        out_shape=[jax.ShapeDtypeStruct((1, 1), jnp.float32)] * 5,
        scratch_shapes=[
            pltpu.VMEM((3, _NP), jnp.int32),
            pltpu.VMEM((1, 1, _NP), jnp.int32),
            pltpu.VMEM((1, 1, _NP), jnp.int32),
            pltpu.VMEM((1, _NP), jnp.int32),
        ],
    )(pred_logits, pred_t, knn, gt, gt_t)

    sbce, sman, spos, sneg, sm = [o[0, 0] for o in outs]
    loss = sbce / (_C * sman)
    loss_pos = spos / sm
    loss_neg = sneg / sm
    return (loss, loss_pos, loss_neg)


# speedup vs baseline: 46.9947x; 1.0923x over previous
import jax
import jax.numpy as jnp
from jax.experimental import pallas as pl
from jax.experimental.pallas import tpu as pltpu

_NP = 2048
_K = 8
_C = (_K - 1) * (_K - 1)
_BLK = 128
_GRID = _NP // _BLK


def _main_kernel(pred_ref, pred_t_ref, knn_ref, gt_ref, gt_t_ref,
                 sbce_ref, sman_ref, spos_ref, sneg_ref, sm_ref,
                 tri_ref, hi3_ref, lo3_ref, valid_ref):
    step = pl.program_id(0)
    zero11 = jnp.zeros((1, 1), dtype=jnp.float32)

    @pl.when(step == 0)
    def _prep():
        a = gt_t_ref[0:1, :]
        b = gt_t_ref[1:2, :]
        c = gt_t_ref[2:3, :]
        p = jnp.minimum(a, b)
        q = jnp.maximum(a, b)
        s0 = jnp.minimum(p, c)
        s2 = jnp.maximum(q, c)
        s1 = a + b + c - s0 - s2
        tri_ref[0:1, :] = s0
        tri_ref[1:2, :] = s1
        tri_ref[2:3, :] = s2
        khi_r = s0 * _NP + s1
        klo_r = s2
        hi3_ref[0:1, 0:1, :] = khi_r.reshape(1, 1, _NP)
        lo3_ref[0:1, 0:1, :] = klo_r.reshape(1, 1, _NP)

        ac = gt_ref[:, 0:1]
        bc = gt_ref[:, 1:2]
        cc = gt_ref[:, 2:3]
        pc = jnp.minimum(ac, bc)
        qc = jnp.maximum(ac, bc)
        s0c = jnp.minimum(pc, cc)
        s2c = jnp.maximum(qc, cc)
        s1c = ac + bc + cc - s0c - s2c
        khi_c = s0c * _NP + s1c
        klo_c = s2c

        u_lane = jax.lax.broadcasted_iota(jnp.int32, (_BLK, _NP), 1)
        t_sub = jax.lax.broadcasted_iota(jnp.int32, (_BLK, _NP), 0)
        for ch in range(_NP // _BLK):
            beg, end = ch * _BLK, (ch + 1) * _BLK
            eq = (khi_c[beg:end, :] == khi_r) & (klo_c[beg:end, :] == klo_r)
            dup = jnp.any(eq & (u_lane < (t_sub + ch * _BLK)), axis=1,
                          keepdims=True)
            valid_ref[0:1, beg:end] = jnp.where(
                dup, 0, 1).astype(jnp.int32).reshape(1, _BLK)

        sbce_ref[:, :] = zero11
        sman_ref[:, :] = zero11
        spos_ref[:, :] = zero11
        sneg_ref[:, :] = zero11
        sm_ref[:, :] = zero11

    v0 = knn_ref[:, 0:1]
    nn_cols = [knn_ref[:, 1 + j:2 + j] for j in range(_K - 1)]
    pairs = [(j, k) for j in range(_K - 1) for k in range(j, _K - 1)]

    s0r = tri_ref[0:1, :]
    s1r = tri_ref[1:2, :]
    s2r = tri_ref[2:3, :]
    vrow = valid_ref[0:1, :]

    e0 = v0 == s0r
    e1 = v0 == s1r
    e2 = v0 == s2r
    in0 = e0 | e1 | e2

    contains = in0 & (vrow > 0)
    angt = jnp.sum(contains.astype(jnp.float32), axis=1, keepdims=True)

    dup_any = jnp.max(
        jnp.where((e0 & e1) | (e1 & e2), 1.0, 0.0), axis=1, keepdims=True)
    gl_cols = []
    for j in range(_K - 1):
        nj = nn_cols[j]
        inj = (nj == s0r) | (nj == s1r) | (nj == s2r)
        both = jnp.max(
            jnp.where(in0 & inj, 1.0, 0.0), axis=1, keepdims=True)
        gl_cols.append(jnp.where(nj == v0, dup_any, both))
    gl = jnp.concatenate(gl_cols, axis=1)

    cand_a = jnp.concatenate([nn_cols[j] for j, _ in pairs], axis=1)
    cand_b = jnp.concatenate([nn_cols[k] for _, k in pairs], axis=1)
    lo = jnp.minimum(cand_a, cand_b)
    hi = jnp.maximum(cand_a, cand_b)
    v0b = jnp.broadcast_to(v0, cand_a.shape)
    c0 = jnp.minimum(lo, v0b)
    c2 = jnp.maximum(hi, v0b)
    c1 = cand_a + cand_b + v0b - c0 - c2
    tk_hi = c0 * _NP + c1
    tk_lo = c2

    m_acc = jnp.zeros(tk_hi.shape, dtype=jnp.bool_)
    t_ch = 512
    for ch in range(_NP // t_ch):
        th = hi3_ref[0:1, 0:1, pl.ds(ch * t_ch, t_ch)]
        tl = lo3_ref[0:1, 0:1, pl.ds(ch * t_ch, t_ch)]
        eq = (tk_hi[:, :, None] == th) & (tk_lo[:, :, None] == tl)
        m_acc = m_acc | jnp.any(eq, axis=2)
    mask28 = m_acc.astype(jnp.float32)

    wrow = jnp.concatenate(
        [jnp.full((1, 1), 1.0 if j == k else 2.0, jnp.float32)
         for j, k in pairs], axis=1)
    npred = jnp.sum(mask28 * wrow, axis=1, keepdims=True)
    manifold = (2.0 * angt == npred).astype(jnp.float32)

    x = pred_ref[:, :]
    bce0 = jnp.maximum(x, 0.0) + jnp.log(1.0 + jnp.exp(-jnp.abs(x)))
    xsym = jnp.concatenate(
        [x[:, j * (_K - 1) + k:j * (_K - 1) + k + 1]
         if j == k else
         (x[:, j * (_K - 1) + k:j * (_K - 1) + k + 1]
          + x[:, k * (_K - 1) + j:k * (_K - 1) + j + 1])
         for j, k in pairs], axis=1)
    row_bce = (jnp.sum(bce0, axis=1, keepdims=True)
               - jnp.sum(xsym * mask28, axis=1, keepdims=True))
    sbce = jnp.sum(row_bce * manifold, keepdims=True).reshape(1, 1)
    sman = jnp.sum(manifold, keepdims=True).reshape(1, 1)

    xk = [pred_t_ref[k] for k in range(_K - 1)]
    pos = jnp.zeros(xk[0].shape, dtype=jnp.float32)
    neg = jnp.zeros(xk[0].shape, dtype=jnp.float32)
    for k in range(_K - 1):
        cnt = jnp.zeros(xk[0].shape, dtype=jnp.int32)
        for kp in range(_K - 1):
            if kp == k:
                continue
            gtk = (xk[kp] > xk[k]) | ((xk[kp] == xk[k]) & (kp < k))
            cnt = cnt + gtk.astype(jnp.int32)
        pos = pos + xk[k] * (cnt == 1).astype(jnp.float32)
        neg = neg + xk[k] * (cnt == 2).astype(jnp.float32)
    mmask = (gl == 1.0).astype(jnp.float32)
    bpos = jnp.maximum(pos, 0.0) - pos + jnp.log(1.0 + jnp.exp(-jnp.abs(pos)))
    bneg = jnp.maximum(neg, 0.0) + jnp.log(1.0 + jnp.exp(-jnp.abs(neg)))
    spos = jnp.sum(bpos * mmask, keepdims=True).reshape(1, 1)
    sneg = jnp.sum(bneg * mmask, keepdims=True).reshape(1, 1)
    sm = jnp.sum(mmask, keepdims=True).reshape(1, 1)

    sbce_ref[:, :] += sbce
    sman_ref[:, :] += sman
    spos_ref[:, :] += spos
    sneg_ref[:, :] += sneg
    sm_ref[:, :] += sm


def kernel(pred_logits, points, knn_indices, gt_triangles):
    del points
    gt = gt_triangles.astype(jnp.int32)
    knn = knn_indices.astype(jnp.int32)
    gt_t = gt.T
    pred_t = pred_logits.reshape(_NP, _K - 1, _K - 1).transpose(2, 0, 1)

    outs = pl.pallas_call(
        _main_kernel,
        grid=(_GRID,),
        in_specs=[
            pl.BlockSpec((_BLK, _C), lambda i: (i, 0)),
            pl.BlockSpec((_K - 1, _BLK, _K - 1), lambda i: (0, i, 0)),
            pl.BlockSpec((_BLK, _K), lambda i: (i, 0)),
            pl.BlockSpec((_NP, 3), lambda i: (0, 0)),
            pl.BlockSpec((3, _NP), lambda i: (0, 0)),
        ],
        out_specs=[pl.BlockSpec((1, 1), lambda i: (0, 0))] * 5,
        out_shape=[jax.ShapeDtypeStruct((1, 1), jnp.float32)] * 5,
        scratch_shapes=[
            pltpu.VMEM((3, _NP), jnp.int32),
            pltpu.VMEM((1, 1, _NP), jnp.int32),
            pltpu.VMEM((1, 1, _NP), jnp.int32),
            pltpu.VMEM((1, _NP), jnp.int32),
        ],
    )(pred_logits, pred_t, knn, gt, gt_t)

    sbce, sman, spos, sneg, sm = [o[0, 0] for o in outs]
    loss = sbce / (_C * sman)
    loss_pos = spos / sm
    loss_neg = sneg / sm
    return (loss, loss_pos, loss_neg)


# speedup vs baseline: 50.0730x; 1.0655x over previous
import jax
import jax.numpy as jnp
from jax.experimental import pallas as pl
from jax.experimental.pallas import tpu as pltpu

_NP = 2048
_K = 8
_C = (_K - 1) * (_K - 1)
_BLK = 128
_GRID = _NP // _BLK


def _main_kernel(pred_ref, pred_t_ref, knn_ref, gt_ref, gt_t_ref,
                 sbce_ref, sman_ref, spos_ref, sneg_ref, sm_ref,
                 tri_ref, hi3_ref, lo3_ref, valid_ref):
    step = pl.program_id(0)
    zero11 = jnp.zeros((1, 1), dtype=jnp.float32)

    @pl.when(step == 0)
    def _prep():
        a = gt_t_ref[0:1, :]
        b = gt_t_ref[1:2, :]
        c = gt_t_ref[2:3, :]
        p = jnp.minimum(a, b)
        q = jnp.maximum(a, b)
        s0 = jnp.minimum(p, c)
        s2 = jnp.maximum(q, c)
        s1 = a + b + c - s0 - s2
        tri_ref[0:1, :] = s0
        tri_ref[1:2, :] = s1
        tri_ref[2:3, :] = s2
        khi_r = s0 * _NP + s1
        klo_r = s2
        hi3_ref[0:1, 0:1, :] = khi_r.reshape(1, 1, _NP)
        lo3_ref[0:1, 0:1, :] = klo_r.reshape(1, 1, _NP)

        ac = gt_ref[:, 0:1]
        bc = gt_ref[:, 1:2]
        cc = gt_ref[:, 2:3]
        pc = jnp.minimum(ac, bc)
        qc = jnp.maximum(ac, bc)
        s0c = jnp.minimum(pc, cc)
        s2c = jnp.maximum(qc, cc)
        s1c = ac + bc + cc - s0c - s2c
        khi_c = s0c * _NP + s1c
        klo_c = s2c

        u_lane = jax.lax.broadcasted_iota(jnp.int32, (_BLK, _NP), 1)
        t_sub = jax.lax.broadcasted_iota(jnp.int32, (_BLK, _NP), 0)
        for ch in range(_NP // _BLK):
            beg, end = ch * _BLK, (ch + 1) * _BLK
            eq = (khi_c[beg:end, :] == khi_r) & (klo_c[beg:end, :] == klo_r)
            dup = jnp.any(eq & (u_lane < (t_sub + ch * _BLK)), axis=1,
                          keepdims=True)
            valid_ref[0:1, beg:end] = jnp.where(
                dup, 0, 1).astype(jnp.int32).reshape(1, _BLK)

        sbce_ref[:, :] = zero11
        sman_ref[:, :] = zero11
        spos_ref[:, :] = zero11
        sneg_ref[:, :] = zero11
        sm_ref[:, :] = zero11

    v0 = knn_ref[:, 0:1]
    nn_cols = [knn_ref[:, 1 + j:2 + j] for j in range(_K - 1)]
    pairs = [(j, k) for j in range(_K - 1) for k in range(j, _K - 1)]

    s0r = tri_ref[0:1, :]
    s1r = tri_ref[1:2, :]
    s2r = tri_ref[2:3, :]
    vrow = valid_ref[0:1, :]

    e0 = v0 == s0r
    e1 = v0 == s1r
    e2 = v0 == s2r
    in0 = e0 | e1 | e2

    contains = in0 & (vrow > 0)
    angt = jnp.sum(contains.astype(jnp.float32), axis=1, keepdims=True)

    dup_any = jnp.max(
        jnp.where((e0 & e1) | (e1 & e2), 1.0, 0.0), axis=1, keepdims=True)
    gl_cols = []
    for j in range(_K - 1):
        nj = nn_cols[j]
        inj = (nj == s0r) | (nj == s1r) | (nj == s2r)
        both = jnp.max(
            jnp.where(in0 & inj, 1.0, 0.0), axis=1, keepdims=True)
        gl_cols.append(jnp.where(nj == v0, dup_any, both))
    gl = jnp.concatenate(gl_cols, axis=1)

    cand_a = jnp.concatenate([nn_cols[j] for j, _ in pairs], axis=1)
    cand_b = jnp.concatenate([nn_cols[k] for _, k in pairs], axis=1)
    lo = jnp.minimum(cand_a, cand_b)
    hi = jnp.maximum(cand_a, cand_b)
    v0b = jnp.broadcast_to(v0, cand_a.shape)
    c0 = jnp.minimum(lo, v0b)
    c2 = jnp.maximum(hi, v0b)
    c1 = cand_a + cand_b + v0b - c0 - c2
    tk_hi = c0 * _NP + c1
    tk_lo = c2

    m_acc = jnp.zeros(tk_hi.shape, dtype=jnp.bool_)
    t_ch = 2048
    for ch in range(_NP // t_ch):
        th = hi3_ref[0:1, 0:1, pl.ds(ch * t_ch, t_ch)]
        tl = lo3_ref[0:1, 0:1, pl.ds(ch * t_ch, t_ch)]
        eq = (tk_hi[:, :, None] == th) & (tk_lo[:, :, None] == tl)
        m_acc = m_acc | jnp.any(eq, axis=2)
    mask28 = m_acc.astype(jnp.float32)

    wrow = jnp.concatenate(
        [jnp.full((1, 1), 1.0 if j == k else 2.0, jnp.float32)
         for j, k in pairs], axis=1)
    npred = jnp.sum(mask28 * wrow, axis=1, keepdims=True)
    manifold = (2.0 * angt == npred).astype(jnp.float32)

    x = pred_ref[:, :]
    bce0 = jnp.maximum(x, 0.0) + jnp.log(1.0 + jnp.exp(-jnp.abs(x)))
    xsym = jnp.concatenate(
        [x[:, j * (_K - 1) + k:j * (_K - 1) + k + 1]
         if j == k else
         (x[:, j * (_K - 1) + k:j * (_K - 1) + k + 1]
          + x[:, k * (_K - 1) + j:k * (_K - 1) + j + 1])
         for j, k in pairs], axis=1)
    row_bce = (jnp.sum(bce0, axis=1, keepdims=True)
               - jnp.sum(xsym * mask28, axis=1, keepdims=True))
    sbce = jnp.sum(row_bce * manifold, keepdims=True).reshape(1, 1)
    sman = jnp.sum(manifold, keepdims=True).reshape(1, 1)

    xk = [pred_t_ref[k] for k in range(_K - 1)]
    pos = jnp.zeros(xk[0].shape, dtype=jnp.float32)
    neg = jnp.zeros(xk[0].shape, dtype=jnp.float32)
    for k in range(_K - 1):
        cnt = jnp.zeros(xk[0].shape, dtype=jnp.int32)
        for kp in range(_K - 1):
            if kp == k:
                continue
            gtk = (xk[kp] > xk[k]) | ((xk[kp] == xk[k]) & (kp < k))
            cnt = cnt + gtk.astype(jnp.int32)
        pos = pos + xk[k] * (cnt == 1).astype(jnp.float32)
        neg = neg + xk[k] * (cnt == 2).astype(jnp.float32)
    mmask = (gl == 1.0).astype(jnp.float32)
    bpos = jnp.maximum(pos, 0.0) - pos + jnp.log(1.0 + jnp.exp(-jnp.abs(pos)))
    bneg = jnp.maximum(neg, 0.0) + jnp.log(1.0 + jnp.exp(-jnp.abs(neg)))
    spos = jnp.sum(bpos * mmask, keepdims=True).reshape(1, 1)
    sneg = jnp.sum(bneg * mmask, keepdims=True).reshape(1, 1)
    sm = jnp.sum(mmask, keepdims=True).reshape(1, 1)

    sbce_ref[:, :] += sbce
    sman_ref[:, :] += sman
    spos_ref[:, :] += spos
    sneg_ref[:, :] += sneg
    sm_ref[:, :] += sm


def kernel(pred_logits, points, knn_indices, gt_triangles):
    del points
    gt = gt_triangles.astype(jnp.int32)
    knn = knn_indices.astype(jnp.int32)
    gt_t = gt.T
    pred_t = pred_logits.reshape(_NP, _K - 1, _K - 1).transpose(2, 0, 1)

    outs = pl.pallas_call(
        _main_kernel,
        grid=(_GRID,),
        in_specs=[
            pl.BlockSpec((_BLK, _C), lambda i: (i, 0)),
            pl.BlockSpec((_K - 1, _BLK, _K - 1), lambda i: (0, i, 0)),
            pl.BlockSpec((_BLK, _K), lambda i: (i, 0)),
            pl.BlockSpec((_NP, 3), lambda i: (0, 0)),
            pl.BlockSpec((3, _NP), lambda i: (0, 0)),
        ],
        out_specs=[pl.BlockSpec((1, 1), lambda i: (0, 0))] * 5,
        out_shape=[jax.ShapeDtypeStruct((1, 1), jnp.float32)] * 5,
        scratch_shapes=[
            pltpu.VMEM((3, _NP), jnp.int32),
            pltpu.VMEM((1, 1, _NP), jnp.int32),
            pltpu.VMEM((1, 1, _NP), jnp.int32),
            pltpu.VMEM((1, _NP), jnp.int32),
        ],
    )(pred_logits, pred_t, knn, gt, gt_t)

    sbce, sman, spos, sneg, sm = [o[0, 0] for o in outs]
    loss = sbce / (_C * sman)
    loss_pos = spos / sm
    loss_neg = sneg / sm
    return (loss, loss_pos, loss_neg)
